# baseline (speedup 1.0000x reference)
import functools

import jax
import jax.numpy as jnp
from jax import lax
from jax.experimental import pallas as pl
from jax.experimental.pallas import tpu as pltpu
from jax.experimental.pallas import tpu_sc as plsc

B = 4
L = 2048
D = 1024
NW = 32
LPW = L // NW
TPW = B * LPW
CHUNK = 16
NCHUNK = TPW // CHUNK
NBUF = 4
PD = 3
PHL = 32
CPP = 8
LBLK = 2
LANES = 16


@jax.jit
def _sc_embed(idx, table, pos_table):
    mesh = plsc.VectorSubcoreMesh(core_axis_name="c", subcore_axis_name="s")

    @functools.partial(
        pl.kernel,
        out_type=jax.ShapeDtypeStruct((B * L, D), jnp.float32),
        mesh=mesh,
        scratch_types=[
            pltpu.VMEM((B, LPW), jnp.int32),
            pltpu.VMEM((NBUF, CHUNK, D), jnp.float32),
            pltpu.VMEM((PHL, D), jnp.float32),
            pltpu.SemaphoreType.DMA((NBUF,)),
            pltpu.SemaphoreType.DMA((NBUF,)),
        ],
    )
    def k(idx_hbm, table_hbm, pos_hbm, out_hbm, idx_v, tok_v, pos_v, sg, so):
        wid = lax.axis_index("s") * 2 + lax.axis_index("c")
        lbase = wid * LPW
        for bb in range(B):
            pltpu.sync_copy(idx_hbm.at[bb, pl.ds(lbase, LPW)], idx_v.at[bb])
        pltpu.sync_copy(pos_hbm.at[pl.ds(lbase, PHL)], pos_v)

        def decomp(c):
            p = c // CPP
            kk = lax.rem(c, CPP)
            bb = kk // LBLK
            loff = p * PHL + lax.rem(kk, LBLK) * CHUNK
            return bb, loff, lax.rem(kk, LBLK) * CHUNK

        def gather(c, b):
            bb, loff, _ = decomp(c)
            ii = idx_v.at[bb].at[pl.ds(loff, CHUNK)]
            return pltpu.make_async_copy(
                table_hbm.at[ii], tok_v.at[b], sg.at[b])

        def out_store(c, b):
            bb, loff, _ = decomp(c)
            row0 = bb * L + lbase + loff
            return pltpu.make_async_copy(
                tok_v.at[b], out_hbm.at[pl.ds(row0, CHUNK)], so.at[b])

        for b in range(PD):
            gather(b, b).start()

        @pl.loop(0, NCHUNK, step=NBUF)
        def _group(c0):
            @pl.when(c0 == CPP)
            def _repos():
                pltpu.sync_copy(
                    pos_hbm.at[pl.ds(lbase + PHL, PHL)], pos_v)

            for b in range(NBUF):
                c = c0 + b
                gather(c, b).wait()
                _, _, pr0 = decomp(c)

                @plsc.parallel_loop(0, CHUNK * D, step=LANES, unroll=8)
                def _elt(i):
                    r = i >> 10
                    col = pl.multiple_of(i & (D - 1), LANES)
                    v = pos_v[pr0 + r, pl.ds(col, LANES)]
                    plsc.addupdate(tok_v.at[b].at[r, pl.ds(col, LANES)], v)

                out_store(c, b).start()

                cp = c + PD
                bp = (b + PD) % NBUF

                @pl.when(cp < NCHUNK)
                def _prefetch():
                    @pl.when(cp >= NBUF)
                    def _drain():
                        out_store(cp - NBUF, bp).wait()

                    gather(cp, bp).start()

        for b in range(NBUF):
            out_store(NCHUNK - NBUF + b, b).wait()

    return k(idx, table, pos_table)


def kernel(x, table, pos_table):
    out = _sc_embed(x.astype(jnp.int32), table, pos_table)
    return out.reshape(B, L, D)


# speedup vs baseline: 1.0171x; 1.0171x over previous
import functools

import jax
import jax.numpy as jnp
from jax import lax
from jax.experimental import pallas as pl
from jax.experimental.pallas import tpu as pltpu
from jax.experimental.pallas import tpu_sc as plsc

B = 4
L = 2048
D = 1024
NW = 32
LPW = L // NW
TPW = B * LPW
CHUNK = 8
NCHUNK = TPW // CHUNK
NBUF = 8
PD = 5
PHL = 32
CPP = 16
LBLK = 4
LANES = 16


@jax.jit
def _sc_embed(idx, table, pos_table):
    mesh = plsc.VectorSubcoreMesh(core_axis_name="c", subcore_axis_name="s")

    @functools.partial(
        pl.kernel,
        out_type=jax.ShapeDtypeStruct((B * L, D), jnp.float32),
        mesh=mesh,
        scratch_types=[
            pltpu.VMEM((B, LPW), jnp.int32),
            pltpu.VMEM((NBUF, CHUNK, D), jnp.float32),
            pltpu.VMEM((PHL, D), jnp.float32),
            pltpu.SemaphoreType.DMA((NBUF,)),
            pltpu.SemaphoreType.DMA((NBUF,)),
        ],
    )
    def k(idx_hbm, table_hbm, pos_hbm, out_hbm, idx_v, tok_v, pos_v, sg, so):
        wid = lax.axis_index("s") * 2 + lax.axis_index("c")
        lbase = wid * LPW
        for bb in range(B):
            pltpu.sync_copy(idx_hbm.at[bb, pl.ds(lbase, LPW)], idx_v.at[bb])
        pltpu.sync_copy(pos_hbm.at[pl.ds(lbase, PHL)], pos_v)

        def decomp(c):
            p = c // CPP
            kk = lax.rem(c, CPP)
            bb = kk // LBLK
            loff = p * PHL + lax.rem(kk, LBLK) * CHUNK
            return bb, loff, lax.rem(kk, LBLK) * CHUNK

        def gather(c, b):
            bb, loff, _ = decomp(c)
            ii = idx_v.at[bb].at[pl.ds(loff, CHUNK)]
            return pltpu.make_async_copy(
                table_hbm.at[ii], tok_v.at[b], sg.at[b])

        def out_store(c, b):
            bb, loff, _ = decomp(c)
            row0 = bb * L + lbase + loff
            return pltpu.make_async_copy(
                tok_v.at[b], out_hbm.at[pl.ds(row0, CHUNK)], so.at[b])

        for b in range(PD):
            gather(b, b).start()

        @pl.loop(0, NCHUNK, step=NBUF)
        def _group(c0):
            @pl.when(c0 == CPP)
            def _repos():
                pltpu.sync_copy(
                    pos_hbm.at[pl.ds(lbase + PHL, PHL)], pos_v)

            for b in range(NBUF):
                c = c0 + b
                gather(c, b).wait()
                _, _, pr0 = decomp(c)

                @plsc.parallel_loop(0, CHUNK * D, step=LANES, unroll=8)
                def _elt(i):
                    r = i >> 10
                    col = pl.multiple_of(i & (D - 1), LANES)
                    v = pos_v[pr0 + r, pl.ds(col, LANES)]
                    plsc.addupdate(tok_v.at[b].at[r, pl.ds(col, LANES)], v)

                out_store(c, b).start()

                cp = c + PD
                bp = (b + PD) % NBUF

                @pl.when(cp < NCHUNK)
                def _prefetch():
                    @pl.when(cp >= NBUF)
                    def _drain():
                        out_store(cp - NBUF, bp).wait()

                    gather(cp, bp).start()

        for b in range(NBUF):
            out_store(NCHUNK - NBUF + b, b).wait()

    return k(idx, table, pos_table)


def kernel(x, table, pos_table):
    out = _sc_embed(x.astype(jnp.int32), table, pos_table)
    return out.reshape(B, L, D)
